# baseline (speedup 1.0000x reference)
import functools

import jax
import jax.numpy as jnp
from jax.experimental import pallas as pl
from jax.experimental.pallas import tpu as pltpu

EMB = 256
NH = 4
DH = 64
NCLS = 8
NROW = NCLS * NH
NACC = NROW + NCLS
NEG = -1e30
NV = 16384
NC = 8192
T = 4096
NTV = NV // T
NTC = NC // T
G0 = NTV
G1 = NTV + NTC
G2 = G1 + NTV
G3 = G2 + NTC


def _dot(a, b, ca, cb):
    return jax.lax.dot_general(
        a, b, (((ca,), (cb,)), ((), ())), preferred_element_type=jnp.float32)


def _mk_init(s, sem2_ref, Wi2_ref, bi2_ref, Ws2_ref, qt_ref, m_ref, l_ref,
             Y_ref):
    f32 = jnp.float32
    sem = sem2_ref[s]
    Wq = Wi2_ref[s, 0:EMB, :]
    Wk = Wi2_ref[s, EMB:2 * EMB, :]
    bq = bi2_ref[s, 0:1, :]
    Q = _dot(sem, Wq, 1, 1) + bq
    rr = jax.lax.broadcasted_iota(jnp.int32, (NROW, NCLS), 0) // NH
    sel = (rr == jax.lax.broadcasted_iota(jnp.int32, (NROW, NCLS), 1)).astype(f32)
    Qexp = _dot(sel, Q, 1, 0)
    hh = jax.lax.broadcasted_iota(jnp.int32, (NROW, EMB), 0) % NH
    ee = jax.lax.broadcasted_iota(jnp.int32, (NROW, EMB), 1) // DH
    Qmask = Qexp * (hh == ee).astype(f32)
    t = _dot(Qmask, Wk, 1, 0)
    qt = _dot(t, Ws2_ref[s], 1, 0)
    qt_ref[...] = qt * (1.0 / 8.0)
    m_ref[...] = jnp.full((NROW, 128), NEG, f32)
    l_ref[...] = jnp.zeros((NACC, 128), f32)
    Y_ref[...] = jnp.zeros((NACC, EMB), f32)


def _mk_acc(xt, clsrow, qt_ref, m_ref, l_ref, Y_ref):
    f32 = jnp.float32
    Tl = xt.shape[0]
    ST = _dot(qt_ref[...], xt, 1, 1)
    ccls = jax.lax.broadcasted_iota(jnp.int32, (NROW, Tl), 0) // NH
    msk = ccls == clsrow
    STm = jnp.where(msk, ST, NEG)
    tmax = jnp.max(STm, axis=1, keepdims=True)
    mold = m_ref[:, 0:1]
    mnew = jnp.maximum(mold, tmax)
    resc = jnp.exp(mold - mnew)
    P = jnp.where(msk, jnp.exp(STm - mnew), 0.0)
    c8 = jax.lax.broadcasted_iota(jnp.int32, (NCLS, Tl), 0)
    P8 = (c8 == clsrow).astype(f32)
    P40 = jnp.concatenate((P, P8), axis=0)
    resc40 = jnp.concatenate((resc, jnp.ones((NCLS, 1), f32)), axis=0)
    l_ref[...] = l_ref[...] * resc40 + jnp.sum(P40, axis=1, keepdims=True)
    Y_ref[...] = Y_ref[...] * resc40 + _dot(P40, xt, 1, 0)
    m_ref[...] = jnp.broadcast_to(mnew, (NROW, 128))


def _mk_fin(s, sem2_ref, Wi2_ref, bi2_ref, Ws2_ref, bs2_ref, Wo2_ref, bo2_ref,
            recW2_ref, recb2_ref, gateW2_ref, gateb2_ref, ng_ref, nb_ref,
            m_ref, l_ref, Y_ref, fused_ref):
    Ws = Ws2_ref[s]
    bs = bs2_ref[s]
    ybar = Y_ref[0:NROW, :] / l_ref[0:NROW, 0:1]
    U = _dot(ybar, Ws, 1, 1) + bs
    Wv = Wi2_ref[s, 2 * EMB:3 * EMB, :]
    bv = bi2_ref[s, 2:3, :]
    Vf = _dot(U, Wv, 1, 1) + bv
    hh2 = jax.lax.broadcasted_iota(jnp.int32, (NROW, EMB), 0) % NH
    ee2 = jax.lax.broadcasted_iota(jnp.int32, (NROW, EMB), 1) // DH
    Vm = Vf * (hh2 == ee2).astype(jnp.float32)
    rr2 = jax.lax.broadcasted_iota(jnp.int32, (NCLS, NROW), 1) // NH
    sel2 = (rr2 == jax.lax.broadcasted_iota(jnp.int32, (NCLS, NROW), 0)).astype(jnp.float32)
    attheads = _dot(sel2, Vm, 1, 0)
    att = _dot(attheads, Wo2_ref[s], 1, 1) + bo2_ref[s]
    old = _dot(Y_ref[NROW:NACC, :] / l_ref[NROW:NACC, 0:1], Ws, 1, 1) + bs
    sem = sem2_ref[s]
    recW = recW2_ref[s]
    new = (_dot(sem, recW[:, 0:EMB], 1, 1)
           + _dot(att, recW[:, EMB:2 * EMB], 1, 1) + recb2_ref[s])
    gW = gateW2_ref[s]
    g = jax.nn.sigmoid(_dot(old, gW[:, 0:EMB], 1, 1)
                       + _dot(new, gW[:, EMB:2 * EMB], 1, 1) + gateb2_ref[s])
    fused = g * old + (1.0 - g) * new
    mu = jnp.mean(fused, axis=1, keepdims=True)
    var = jnp.mean((fused - mu) ** 2, axis=1, keepdims=True)
    fused_ref[...] = ((fused - mu) / jnp.sqrt(var + 1e-5) * ng_ref[...]
                      + nb_ref[...])


def _merged_kernel(xv_ref, xc_ref, clsv_ref, clsc_ref, sem2_ref, Wi2_ref,
                   bi2_ref, Ws2_ref, bs2_ref, Wo2_ref, bo2_ref, recW2_ref,
                   recb2_ref, gateW2_ref, gateb2_ref, ng_ref, nb_ref,
                   outv_ref, outc_ref,
                   qt_ref, m_ref, l_ref, Y_ref, fusedv_ref, fusedc_ref,
                   xsave_ref):
    i = pl.program_id(0)

    @pl.when(i == 0)
    def _initv():
        _mk_init(0, sem2_ref, Wi2_ref, bi2_ref, Ws2_ref, qt_ref, m_ref,
                 l_ref, Y_ref)

    @pl.when(i < G0)
    def _accv():
        xt = xv_ref[...]
        xsave_ref[pl.ds(i * T, T), :] = xt
        _mk_acc(xt, clsv_ref[i], qt_ref, m_ref, l_ref, Y_ref)

    @pl.when(i == G0 - 1)
    def _finv():
        _mk_fin(0, sem2_ref, Wi2_ref, bi2_ref, Ws2_ref, bs2_ref, Wo2_ref,
                bo2_ref, recW2_ref, recb2_ref, gateW2_ref, gateb2_ref,
                ng_ref, nb_ref, m_ref, l_ref, Y_ref, fusedv_ref)

    @pl.when(i == G0)
    def _initc():
        _mk_init(1, sem2_ref, Wi2_ref, bi2_ref, Ws2_ref, qt_ref, m_ref,
                 l_ref, Y_ref)

    @pl.when(jnp.logical_and(i >= G0, i < G1))
    def _accc():
        _mk_acc(xc_ref[...], clsc_ref[i - G0], qt_ref, m_ref, l_ref, Y_ref)

    @pl.when(i == G1 - 1)
    def _finc():
        _mk_fin(1, sem2_ref, Wi2_ref, bi2_ref, Ws2_ref, bs2_ref, Wo2_ref,
                bo2_ref, recW2_ref, recb2_ref, gateW2_ref, gateb2_ref,
                ng_ref, nb_ref, m_ref, l_ref, Y_ref, fusedc_ref)

    @pl.when(jnp.logical_and(i >= G1, i < G2))
    def _applyv():
        j = i - G1
        xt = xsave_ref[pl.ds(j * T, T), :]
        clsrow = clsv_ref[j]
        c8 = jax.lax.broadcasted_iota(jnp.int32, (NCLS, T), 0)
        P8 = (c8 == clsrow).astype(jnp.float32)
        g = _dot(P8, fusedv_ref[...], 0, 0)
        outv_ref[...] = g * xt

    @pl.when(i >= G2)
    def _applyc():
        j = i - G2
        xt = xc_ref[...]
        clsrow = clsc_ref[j]
        c8 = jax.lax.broadcasted_iota(jnp.int32, (NCLS, T), 0)
        P8 = (c8 == clsrow).astype(jnp.float32)
        g = _dot(P8, fusedc_ref[...], 0, 0)
        outc_ref[...] = g * xt


def kernel(v, c, v_sem, c_sem, params, v_class, c_class):
    p = params
    st = lambda a, b: jnp.stack((a, b))
    sem2 = st(v_sem, c_sem)
    Wi2 = st(p['av_Wi'], p['ac_Wi'])
    bi2 = st(p['av_bi'].reshape(3, EMB), p['ac_bi'].reshape(3, EMB))
    Ws2 = st(p['send_var_W'], p['send_con_W'])
    bs2 = st(p['send_var_b'].reshape(1, EMB), p['send_con_b'].reshape(1, EMB))
    Wo2 = st(p['av_Wo'], p['ac_Wo'])
    bo2 = st(p['av_bo'].reshape(1, EMB), p['ac_bo'].reshape(1, EMB))
    recW2 = st(p['rec_var_W'], p['rec_con_W'])
    recb2 = st(p['rec_var_b'].reshape(1, EMB), p['rec_con_b'].reshape(1, EMB))
    gateW2 = st(p['gate_v_W'], p['gate_c_W'])
    gateb2 = st(p['gate_v_b'].reshape(1, EMB), p['gate_c_b'].reshape(1, EMB))
    ng = p['norm_g'].reshape(1, EMB)
    nb = p['norm_b'].reshape(1, EMB)
    clsv = v_class.astype(jnp.int32).reshape(NTV, 1, T)
    clsc = c_class.astype(jnp.int32).reshape(NTC, 1, T)

    full = lambda s: pl.BlockSpec(s, lambda i: (0,) * len(s))

    outv, outc = pl.pallas_call(
        _merged_kernel,
        grid=(G3,),
        in_specs=[
            pl.BlockSpec((T, EMB), lambda i: (jnp.minimum(i, G0 - 1), 0)),
            pl.BlockSpec((T, EMB), lambda i: (
                jnp.where(i < G2, jnp.clip(i - G0, 0, NTC - 1), i - G2), 0)),
            full((NTV, 1, T)), full((NTC, 1, T)),
            full((2, NCLS, EMB)), full((2, 3 * EMB, EMB)), full((2, 3, EMB)),
            full((2, EMB, EMB)), full((2, 1, EMB)),
            full((2, EMB, EMB)), full((2, 1, EMB)),
            full((2, EMB, 2 * EMB)), full((2, 1, EMB)),
            full((2, EMB, 2 * EMB)), full((2, 1, EMB)),
            full((1, EMB)), full((1, EMB)),
        ],
        out_specs=[
            pl.BlockSpec((T, EMB), lambda i: (jnp.clip(i - G1, 0, NTV - 1), 0)),
            pl.BlockSpec((T, EMB), lambda i: (jnp.clip(i - G2, 0, NTC - 1), 0)),
        ],
        out_shape=[
            jax.ShapeDtypeStruct((NV, EMB), jnp.float32),
            jax.ShapeDtypeStruct((NC, EMB), jnp.float32),
        ],
        scratch_shapes=[
            pltpu.VMEM((NROW, EMB), jnp.float32),
            pltpu.VMEM((NROW, 128), jnp.float32),
            pltpu.VMEM((NACC, 128), jnp.float32),
            pltpu.VMEM((NACC, EMB), jnp.float32),
            pltpu.VMEM((NCLS, EMB), jnp.float32),
            pltpu.VMEM((NCLS, EMB), jnp.float32),
            pltpu.VMEM((NV, EMB), jnp.float32),
        ],
    )(v, c, clsv, clsc, sem2, Wi2, bi2, Ws2, bs2, Wo2, bo2, recW2, recb2,
      gateW2, gateb2, ng, nb)
    return outv, outc


# speedup vs baseline: 1.1994x; 1.1994x over previous
import functools

import jax
import jax.numpy as jnp
from jax.experimental import pallas as pl
from jax.experimental.pallas import tpu as pltpu

EMB = 256
NH = 4
DH = 64
NCLS = 8
NROW = NCLS * NH
NACC = NROW + NCLS
NEG = -1e30


def _dot(a, b, ca, cb):
    return jax.lax.dot_general(
        a, b, (((ca,), (cb,)), ((), ())), preferred_element_type=jnp.float32)


def _side_kernel(nt, x_ref, cls_ref, sem_ref, Wi_ref, bi_ref, Ws_ref, bs_ref,
                 Wo_ref, bo_ref, recW_ref, recb_ref, gateW_ref, gateb_ref,
                 ng_ref, nb_ref, out_ref,
                 qt_ref, m_ref, l_ref, Y_ref, fused_ref, xsave_ref):
    i = pl.program_id(0)
    T = x_ref.shape[0]
    f32 = jnp.float32

    @pl.when(i == 0)
    def _init():
        sem = sem_ref[...]
        Wq = Wi_ref[0:EMB, :]
        Wk = Wi_ref[EMB:2 * EMB, :]
        bq = bi_ref[0:1, :]
        Q = _dot(sem, Wq, 1, 1) + bq
        rr = jax.lax.broadcasted_iota(jnp.int32, (NROW, NCLS), 0) // NH
        sel = (rr == jax.lax.broadcasted_iota(jnp.int32, (NROW, NCLS), 1)).astype(f32)
        Qexp = _dot(sel, Q, 1, 0)
        hh = jax.lax.broadcasted_iota(jnp.int32, (NROW, EMB), 0) % NH
        ee = jax.lax.broadcasted_iota(jnp.int32, (NROW, EMB), 1) // DH
        Qmask = Qexp * (hh == ee).astype(f32)
        t = _dot(Qmask, Wk, 1, 0)
        qt = _dot(t, Ws_ref[...], 1, 0)
        qt_ref[...] = qt * (1.0 / 8.0)
        m_ref[...] = jnp.full((NROW, 128), NEG, f32)
        l_ref[...] = jnp.zeros((NACC, 128), f32)
        Y_ref[...] = jnp.zeros((NACC, EMB), f32)

    @pl.when(i < nt)
    def _acc():
        xt = x_ref[...]
        xsave_ref[pl.ds(i * T, T), :] = xt
        clsrow = cls_ref[i]
        ST = _dot(qt_ref[...], xt, 1, 1)
        ccls = jax.lax.broadcasted_iota(jnp.int32, (NROW, T), 0) // NH
        msk = ccls == clsrow
        STm = jnp.where(msk, ST, NEG)
        tmax = jnp.max(STm, axis=1, keepdims=True)
        mold = m_ref[:, 0:1]
        mnew = jnp.maximum(mold, tmax)
        resc = jnp.exp(mold - mnew)
        P = jnp.where(msk, jnp.exp(STm - mnew), 0.0)
        c8 = jax.lax.broadcasted_iota(jnp.int32, (NCLS, T), 0)
        P8 = (c8 == clsrow).astype(f32)
        P40 = jnp.concatenate((P, P8), axis=0)
        resc40 = jnp.concatenate((resc, jnp.ones((NCLS, 1), f32)), axis=0)
        l_ref[...] = l_ref[...] * resc40 + jnp.sum(P40, axis=1, keepdims=True)
        Y_ref[...] = Y_ref[...] * resc40 + _dot(P40, xt, 1, 0)
        m_ref[...] = jnp.broadcast_to(mnew, (NROW, 128))

    @pl.when(i == nt - 1)
    def _fin():
        Ws = Ws_ref[...]
        bs = bs_ref[...]
        ybar = Y_ref[0:NROW, :] / l_ref[0:NROW, 0:1]
        U = _dot(ybar, Ws, 1, 1) + bs
        Wv = Wi_ref[2 * EMB:3 * EMB, :]
        bv = bi_ref[2:3, :]
        Vf = _dot(U, Wv, 1, 1) + bv
        hh2 = jax.lax.broadcasted_iota(jnp.int32, (NROW, EMB), 0) % NH
        ee2 = jax.lax.broadcasted_iota(jnp.int32, (NROW, EMB), 1) // DH
        Vm = Vf * (hh2 == ee2).astype(jnp.float32)
        rr2 = jax.lax.broadcasted_iota(jnp.int32, (NCLS, NROW), 1) // NH
        sel2 = (rr2 == jax.lax.broadcasted_iota(jnp.int32, (NCLS, NROW), 0)).astype(jnp.float32)
        attheads = _dot(sel2, Vm, 1, 0)
        att = _dot(attheads, Wo_ref[...], 1, 1) + bo_ref[...]
        old = _dot(Y_ref[NROW:NACC, :] / l_ref[NROW:NACC, 0:1], Ws, 1, 1) + bs
        sem = sem_ref[...]
        recW = recW_ref[...]
        new = (_dot(sem, recW[:, 0:EMB], 1, 1)
               + _dot(att, recW[:, EMB:2 * EMB], 1, 1) + recb_ref[...])
        gW = gateW_ref[...]
        g = jax.nn.sigmoid(_dot(old, gW[:, 0:EMB], 1, 1)
                           + _dot(new, gW[:, EMB:2 * EMB], 1, 1) + gateb_ref[...])
        fused = g * old + (1.0 - g) * new
        mu = jnp.mean(fused, axis=1, keepdims=True)
        var = jnp.mean((fused - mu) ** 2, axis=1, keepdims=True)
        fused_ref[...] = ((fused - mu) / jnp.sqrt(var + 1e-5) * ng_ref[...]
                          + nb_ref[...])

    @pl.when(i >= nt)
    def _apply():
        j = i - nt
        xt = xsave_ref[pl.ds(j * T, T), :]
        clsrow = cls_ref[j]
        c8 = jax.lax.broadcasted_iota(jnp.int32, (NCLS, T), 0)
        P8 = (c8 == clsrow).astype(jnp.float32)
        g = _dot(P8, fused_ref[...], 0, 0)
        out_ref[...] = g * xt


def _side(x, sem, Wi, bi, Ws, bs, Wo, bo, recW, recb, gateW, gateb, ng, nb,
          cls, T):
    N = x.shape[0]
    nt = N // T
    cls3 = cls.astype(jnp.int32).reshape(nt, 1, T)
    bi3 = bi.reshape(3, EMB)
    row = lambda a: a.reshape(1, EMB)
    full = lambda s: pl.BlockSpec(s, lambda i: (0,) * len(s))

    def tile_map(i):
        return (jnp.minimum(i, nt - 1), 0)

    def out_map(i):
        return (jnp.maximum(i - nt, 0), 0)

    out = pl.pallas_call(
        functools.partial(_side_kernel, nt),
        grid=(2 * nt,),
        in_specs=[
            pl.BlockSpec((T, EMB), tile_map),
            full((nt, 1, T)),
            full((NCLS, EMB)), full((3 * EMB, EMB)), full((3, EMB)),
            full((EMB, EMB)), full((1, EMB)),
            full((EMB, EMB)), full((1, EMB)),
            full((EMB, 2 * EMB)), full((1, EMB)),
            full((EMB, 2 * EMB)), full((1, EMB)),
            full((1, EMB)), full((1, EMB)),
        ],
        out_specs=pl.BlockSpec((T, EMB), out_map),
        out_shape=jax.ShapeDtypeStruct((N, EMB), jnp.float32),
        scratch_shapes=[
            pltpu.VMEM((NROW, EMB), jnp.float32),
            pltpu.VMEM((NROW, 128), jnp.float32),
            pltpu.VMEM((NACC, 128), jnp.float32),
            pltpu.VMEM((NACC, EMB), jnp.float32),
            pltpu.VMEM((NCLS, EMB), jnp.float32),
            pltpu.VMEM((N, EMB), jnp.float32),
        ],
    )(x, cls3, sem, Wi, bi3, Ws, row(bs), Wo, row(bo), recW, row(recb),
      gateW, row(gateb), row(ng), row(nb))
    return out


def kernel(v, c, v_sem, c_sem, params, v_class, c_class):
    p = params
    v_upd = _side(v, v_sem, p['av_Wi'], p['av_bi'], p['send_var_W'],
                  p['send_var_b'], p['av_Wo'], p['av_bo'], p['rec_var_W'],
                  p['rec_var_b'], p['gate_v_W'], p['gate_v_b'], p['norm_g'],
                  p['norm_b'], v_class, 2048)
    c_upd = _side(c, c_sem, p['ac_Wi'], p['ac_bi'], p['send_con_W'],
                  p['send_con_b'], p['ac_Wo'], p['ac_bo'], p['rec_con_W'],
                  p['rec_con_b'], p['gate_c_W'], p['gate_c_b'], p['norm_g'],
                  p['norm_b'], c_class, 2048)
    return v_upd, c_upd


# speedup vs baseline: 1.3521x; 1.1273x over previous
import functools

import jax
import jax.numpy as jnp
from jax.experimental import pallas as pl
from jax.experimental.pallas import tpu as pltpu

EMB = 256
NH = 4
DH = 64
NCLS = 8
NROW = NCLS * NH
NACC = NROW + NCLS
NEG = -1e30


def _dot(a, b, ca, cb):
    return jax.lax.dot_general(
        a, b, (((ca,), (cb,)), ((), ())), preferred_element_type=jnp.float32)


def _side_kernel(nt, x_ref, cls_ref, sem_ref, Wi_ref, bi_ref, Ws_ref, bs_ref,
                 Wo_ref, bo_ref, recW_ref, recb_ref, gateW_ref, gateb_ref,
                 ng_ref, nb_ref, out_ref,
                 qt_ref, m_ref, l_ref, Y_ref, fused_ref, xsave_ref):
    i = pl.program_id(0)
    T = x_ref.shape[0]
    f32 = jnp.float32

    @pl.when(i == 0)
    def _init():
        sem = sem_ref[...]
        Wq = Wi_ref[0:EMB, :]
        Wk = Wi_ref[EMB:2 * EMB, :]
        bq = bi_ref[0:1, :]
        Q = _dot(sem, Wq, 1, 1) + bq
        rr = jax.lax.broadcasted_iota(jnp.int32, (NROW, NCLS), 0) // NH
        sel = (rr == jax.lax.broadcasted_iota(jnp.int32, (NROW, NCLS), 1)).astype(f32)
        Qexp = _dot(sel, Q, 1, 0)
        hh = jax.lax.broadcasted_iota(jnp.int32, (NROW, EMB), 0) % NH
        ee = jax.lax.broadcasted_iota(jnp.int32, (NROW, EMB), 1) // DH
        Qmask = Qexp * (hh == ee).astype(f32)
        t = _dot(Qmask, Wk, 1, 0)
        qt = _dot(t, Ws_ref[...], 1, 0)
        qt_ref[...] = qt * (1.0 / 8.0)
        m_ref[...] = jnp.full((NROW, 128), NEG, f32)
        l_ref[...] = jnp.zeros((NACC, 128), f32)
        Y_ref[...] = jnp.zeros((NACC, EMB), f32)

    @pl.when(i < nt)
    def _acc():
        xt = x_ref[...]
        xsave_ref[pl.ds(i * T, T), :] = xt
        clsrow = cls_ref[i]
        ST = _dot(qt_ref[...], xt, 1, 1)
        ccls = jax.lax.broadcasted_iota(jnp.int32, (NROW, T), 0) // NH
        msk = ccls == clsrow
        STm = jnp.where(msk, ST, NEG)
        tmax = jnp.max(STm, axis=1, keepdims=True)
        mold = m_ref[:, 0:1]
        mnew = jnp.maximum(mold, tmax)
        resc = jnp.exp(mold - mnew)
        P = jnp.where(msk, jnp.exp(STm - mnew), 0.0)
        c8 = jax.lax.broadcasted_iota(jnp.int32, (NCLS, T), 0)
        P8 = (c8 == clsrow).astype(f32)
        P40 = jnp.concatenate((P, P8), axis=0)
        resc40 = jnp.concatenate((resc, jnp.ones((NCLS, 1), f32)), axis=0)
        l_ref[...] = l_ref[...] * resc40 + jnp.sum(P40, axis=1, keepdims=True)
        Y_ref[...] = Y_ref[...] * resc40 + _dot(P40, xt, 1, 0)
        m_ref[...] = jnp.broadcast_to(mnew, (NROW, 128))

    @pl.when(i == nt - 1)
    def _fin():
        Ws = Ws_ref[...]
        bs = bs_ref[...]
        ybar = Y_ref[0:NROW, :] / l_ref[0:NROW, 0:1]
        U = _dot(ybar, Ws, 1, 1) + bs
        Wv = Wi_ref[2 * EMB:3 * EMB, :]
        bv = bi_ref[2:3, :]
        Vf = _dot(U, Wv, 1, 1) + bv
        hh2 = jax.lax.broadcasted_iota(jnp.int32, (NROW, EMB), 0) % NH
        ee2 = jax.lax.broadcasted_iota(jnp.int32, (NROW, EMB), 1) // DH
        Vm = Vf * (hh2 == ee2).astype(jnp.float32)
        rr2 = jax.lax.broadcasted_iota(jnp.int32, (NCLS, NROW), 1) // NH
        sel2 = (rr2 == jax.lax.broadcasted_iota(jnp.int32, (NCLS, NROW), 0)).astype(jnp.float32)
        attheads = _dot(sel2, Vm, 1, 0)
        att = _dot(attheads, Wo_ref[...], 1, 1) + bo_ref[...]
        old = _dot(Y_ref[NROW:NACC, :] / l_ref[NROW:NACC, 0:1], Ws, 1, 1) + bs
        sem = sem_ref[...]
        recW = recW_ref[...]
        new = (_dot(sem, recW[:, 0:EMB], 1, 1)
               + _dot(att, recW[:, EMB:2 * EMB], 1, 1) + recb_ref[...])
        gW = gateW_ref[...]
        g = jax.nn.sigmoid(_dot(old, gW[:, 0:EMB], 1, 1)
                           + _dot(new, gW[:, EMB:2 * EMB], 1, 1) + gateb_ref[...])
        fused = g * old + (1.0 - g) * new
        mu = jnp.mean(fused, axis=1, keepdims=True)
        var = jnp.mean((fused - mu) ** 2, axis=1, keepdims=True)
        fused_ref[...] = ((fused - mu) / jnp.sqrt(var + 1e-5) * ng_ref[...]
                          + nb_ref[...])

    @pl.when(i >= nt)
    def _apply():
        j = i - nt
        xt = xsave_ref[pl.ds(j * T, T), :]
        clsrow = cls_ref[j]
        c8 = jax.lax.broadcasted_iota(jnp.int32, (NCLS, T), 0)
        P8 = (c8 == clsrow).astype(jnp.float32)
        g = _dot(P8, fused_ref[...], 0, 0)
        out_ref[...] = g * xt


def _side(x, sem, Wi, bi, Ws, bs, Wo, bo, recW, recb, gateW, gateb, ng, nb,
          cls, T):
    N = x.shape[0]
    nt = N // T
    cls3 = cls.astype(jnp.int32).reshape(nt, 1, T)
    bi3 = bi.reshape(3, EMB)
    row = lambda a: a.reshape(1, EMB)
    full = lambda s: pl.BlockSpec(s, lambda i: (0,) * len(s))

    def tile_map(i):
        return (jnp.minimum(i, nt - 1), 0)

    def out_map(i):
        return (jnp.maximum(i - nt, 0), 0)

    out = pl.pallas_call(
        functools.partial(_side_kernel, nt),
        grid=(2 * nt,),
        in_specs=[
            pl.BlockSpec((T, EMB), tile_map),
            full((nt, 1, T)),
            full((NCLS, EMB)), full((3 * EMB, EMB)), full((3, EMB)),
            full((EMB, EMB)), full((1, EMB)),
            full((EMB, EMB)), full((1, EMB)),
            full((EMB, 2 * EMB)), full((1, EMB)),
            full((EMB, 2 * EMB)), full((1, EMB)),
            full((1, EMB)), full((1, EMB)),
        ],
        out_specs=pl.BlockSpec((T, EMB), out_map),
        out_shape=jax.ShapeDtypeStruct((N, EMB), jnp.float32),
        scratch_shapes=[
            pltpu.VMEM((NROW, EMB), jnp.float32),
            pltpu.VMEM((NROW, 128), jnp.float32),
            pltpu.VMEM((NACC, 128), jnp.float32),
            pltpu.VMEM((NACC, EMB), jnp.float32),
            pltpu.VMEM((NCLS, EMB), jnp.float32),
            pltpu.VMEM((N, EMB), jnp.float32),
        ],
    )(x, cls3, sem, Wi, bi3, Ws, row(bs), Wo, row(bo), recW, row(recb),
      gateW, row(gateb), row(ng), row(nb))
    return out


def kernel(v, c, v_sem, c_sem, params, v_class, c_class):
    p = params
    v_upd = _side(v, v_sem, p['av_Wi'], p['av_bi'], p['send_var_W'],
                  p['send_var_b'], p['av_Wo'], p['av_bo'], p['rec_var_W'],
                  p['rec_var_b'], p['gate_v_W'], p['gate_v_b'], p['norm_g'],
                  p['norm_b'], v_class, 4096)
    c_upd = _side(c, c_sem, p['ac_Wi'], p['ac_bi'], p['send_con_W'],
                  p['send_con_b'], p['ac_Wo'], p['ac_bo'], p['rec_con_W'],
                  p['rec_con_b'], p['gate_c_W'], p['gate_c_b'], p['norm_g'],
                  p['norm_b'], c_class, 4096)
    return v_upd, c_upd
